# baseline (speedup 1.0000x reference)
import functools

import jax
import jax.numpy as jnp
from jax import lax
from jax.experimental import pallas as pl
from jax.experimental.pallas import tpu as pltpu
from jax.experimental.pallas import tpu_sc as plsc

EMBED = 16
FIELDS_USED = 9


@functools.cache
def _make_gather(batch: int, vocab: int):
    nc, ns = 2, 16
    nw = nc * ns
    units = FIELDS_USED * EMBED * 2
    assert units % nw == 0
    u_per_w = units // nw
    half = batch // 2
    mesh = plsc.VectorSubcoreMesh(core_axis_name="c", subcore_axis_name="s")

    @functools.partial(
        pl.kernel,
        mesh=mesh,
        out_type=jax.ShapeDtypeStruct((FIELDS_USED, EMBED, batch), jnp.float32),
        scratch_types=[
            pltpu.VMEM((half,), jnp.int32),
            pltpu.VMEM((half,), jnp.float32),
            pltpu.SemaphoreType.DMA,
        ],
    )
    def gather_kernel(xt_hbm, tflat_hbm, out_hbm, idx_v, val_v, sem):
        wid = lax.axis_index("s") * nc + lax.axis_index("c")

        def unit(k, _):
            ug = wid * u_per_w + k
            f = ug // (EMBED * 2)
            r = ug % (EMBED * 2)
            d = r // 2
            h = r % 2
            pltpu.sync_copy(xt_hbm.at[f, pl.ds(h * half, half)], idx_v)
            seg = tflat_hbm.at[pl.ds(d * vocab, vocab)]
            pltpu.async_copy(seg.at[idx_v], val_v, sem).wait()
            pltpu.sync_copy(val_v, out_hbm.at[f, d, pl.ds(h * half, half)])
            return ()

        lax.fori_loop(0, u_per_w, unit, (), unroll=False)

    return gather_kernel


def kernel(x, table):
    batch = x.shape[0]
    vocab = table.shape[0]
    xt = x.T[:FIELDS_USED].astype(jnp.int32)
    tflat = table.T.reshape(-1)
    out = _make_gather(batch, vocab)(xt, tflat)
    return out.transpose(2, 0, 1)
